# baseline (speedup 1.0000x reference)
import functools
import math

import jax
import jax.numpy as jnp
from jax import lax
from jax.experimental import pallas as pl
from jax.experimental.pallas import tpu as pltpu
from jax.experimental.pallas import tpu_sc as plsc

D_MODEL = 2048
B, S = 4, 2048
NTOK = B * S
SCALE = math.sqrt(D_MODEL)

NC, NS, L = 2, 16, 16
NW = NC * NS
TPW = NTOK // NW
K = 8
CHUNKS = TPW // K


def _twin_emb_body(ut, up, us, vt, vp, vs, t1, p1, s1, t2, p2, s2,
                   out_u, out_v, itok, ipos, iseg, acc, bbuf, cbuf, sem):
    wid = lax.axis_index("s") * NC + lax.axis_index("c")
    base = wid * TPW
    crow = wid * CHUNKS

    for (tix, pix, six, ttab, ptab, stab, out) in (
        (ut, up, us, t1, p1, s1, out_u),
        (vt, vp, vs, t2, p2, s2, out_v),
    ):
        pltpu.sync_copy(tix.at[pl.ds(crow, CHUNKS)], itok)
        pltpu.sync_copy(pix.at[pl.ds(crow, CHUNKS)], ipos)
        pltpu.sync_copy(six.at[pl.ds(crow, CHUNKS)], iseg)

        def chunk(c, carry):
            ca = pltpu.async_copy(ttab.at[itok.at[c]], acc, sem)
            cb = pltpu.async_copy(ptab.at[ipos.at[c]], bbuf, sem)
            cc = pltpu.async_copy(stab.at[iseg.at[c]], cbuf, sem)
            ca.wait()
            cb.wait()
            cc.wait()

            def col(i, carry2):
                sl = pl.ds(pl.multiple_of(i * L, L), L)
                for r in range(K):
                    acc[r, sl] = (acc[r, sl] + bbuf[r, sl] + cbuf[r, sl]) * SCALE
                return carry2

            lax.fori_loop(0, D_MODEL // L, col, 0)
            row0 = pl.multiple_of(base + c * K, K)
            pltpu.sync_copy(acc, out.at[pl.ds(row0, K)])
            return carry

        lax.fori_loop(0, CHUNKS, chunk, 0)


@jax.jit
def _twin_emb(ut, up, us, vt, vp, vs, t1, p1, s1, t2, p2, s2):
    mesh = plsc.VectorSubcoreMesh(core_axis_name="c", subcore_axis_name="s")
    f = functools.partial(
        pl.kernel,
        out_type=(
            jax.ShapeDtypeStruct((NTOK, D_MODEL), jnp.float32),
            jax.ShapeDtypeStruct((NTOK, D_MODEL), jnp.float32),
        ),
        mesh=mesh,
        scratch_types=[
            pltpu.VMEM((NW * CHUNKS // NW, K), jnp.int32),
            pltpu.VMEM((NW * CHUNKS // NW, K), jnp.int32),
            pltpu.VMEM((NW * CHUNKS // NW, K), jnp.int32),
            pltpu.VMEM((K, D_MODEL), jnp.float32),
            pltpu.VMEM((K, D_MODEL), jnp.float32),
            pltpu.VMEM((K, D_MODEL), jnp.float32),
            pltpu.SemaphoreType.DMA,
        ],
    )(_twin_emb_body)
    return f(ut, up, us, vt, vp, vs, t1, p1, s1, t2, p2, s2)


def kernel(u_tok, u_pos, u_seg, v_tok, v_pos, v_seg,
           tok1, pos1, seg1, tok2, pos2, seg2):
    def prep(ix):
        return ix.reshape(NTOK // K, K).astype(jnp.int32)

    out_u, out_v = _twin_emb(
        prep(u_tok), prep(u_pos), prep(u_seg),
        prep(v_tok), prep(v_pos), prep(v_seg),
        tok1, pos1, seg1, tok2, pos2, seg2)
    return (out_u.reshape(B, S, D_MODEL), out_v.reshape(B, S, D_MODEL))
